# Initial kernel scaffold; baseline (speedup 1.0000x reference)
import jax
import jax.numpy as jnp
from jax.experimental import pallas as pl


def kernel(mlvl_feats, points, ori_points, img, lidar2camera, lidar2image, cam_intrinsic, cam_2_lidar, img_aug_matrix, lidar_aug_matrix, img_metas, conv_w, conv_b, bn_gamma, bn_beta):
    raise NotImplementedError("write your pallas kernel here")


# speedup vs baseline: 4.5616x; 4.5616x over previous
import functools
import jax
import jax.numpy as jnp
from jax import lax
from jax.experimental import pallas as pl
from jax.experimental.pallas import tpu as pltpu
from jax.experimental.pallas import tpu_sc as plsc

_NX, _NY, _NZ = 200, 200, 4
_NCAM, _CIN, _FH, _FW = 6, 256, 64, 176
_COUT = 80
_NXY = _NX * _NY
_NVOX = _NXY * _NZ
_TROWS = _NCAM * _FH * _FW


def _round_ne(x):
    return lax.round(x, lax.RoundingMethod.TO_NEAREST_EVEN)


def _proj_kernel(par_ref, p2i_ref, p2_ref, idx_ref, w_ref):
    idx_sel = jnp.zeros(idx_ref.shape, jnp.int32)
    valid_any = jnp.zeros(idx_ref.shape, jnp.bool_)
    for j in range(_NCAM):
        px = p2i_ref[j, 0]
        py = p2i_ref[j, 1]
        pz = p2i_ref[j, 2]
        u = px / pz + par_ref[2 * j]
        v = py / pz + par_ref[2 * j + 1]
        uf = _round_ne(u * 0.25)
        vf = _round_ne(v * 0.25)
        val = ((uf >= 0.0) & (vf >= 0.0) & (uf < float(_FW))
               & (vf < float(_FH)) & (pz > 0.0))
        uu = jnp.clip(uf, 0.0, float(_FW - 1)).astype(jnp.int32)
        vv = jnp.clip(vf, 0.0, float(_FH - 1)).astype(jnp.int32)
        ind = (j * _FH + vv) * _FW + uu
        idx_sel = jnp.where(val, ind, idx_sel)
        valid_any = valid_any | val
    idx_ref[...] = idx_sel
    w_ref[...] = jnp.where(valid_any, p2_ref[...], 0.0)


def _project(par, p2i, p2):
    br = 40
    grid = _NX // br
    return pl.pallas_call(
        _proj_kernel,
        grid=(grid,),
        in_specs=[
            pl.BlockSpec(memory_space=pltpu.SMEM),
            pl.BlockSpec((_NCAM, 3, br, _NY * _NZ), lambda i: (0, 0, i, 0)),
            pl.BlockSpec((br, _NY * _NZ), lambda i: (i, 0)),
        ],
        out_specs=[
            pl.BlockSpec((br, _NY * _NZ), lambda i: (i, 0)),
            pl.BlockSpec((br, _NY * _NZ), lambda i: (i, 0)),
        ],
        out_shape=[
            jax.ShapeDtypeStruct((_NX, _NY * _NZ), jnp.int32),
            jax.ShapeDtypeStruct((_NX, _NY * _NZ), jnp.float32),
        ],
    )(par, p2i, p2)


_CB = 512
_NB = (_FH * _FW) // _CB
_TD = 128


def _table_kernel(w_ref, f_ref, out_ref):
    out_ref[...] = lax.dot_general(
        f_ref[0], w_ref[...], (((0,), (1,)), ((), ())),
        preferred_element_type=jnp.float32)


def _make_table(conv_w_pad, feat):
    return pl.pallas_call(
        _table_kernel,
        grid=(_NCAM, _NB),
        in_specs=[
            pl.BlockSpec((_TD, _CIN), lambda j, b: (0, 0)),
            pl.BlockSpec((1, _CIN, _CB), lambda j, b: (j, 0, b)),
        ],
        out_specs=pl.BlockSpec((_CB, _TD), lambda j, b: (j * _NB + b, 0)),
        out_shape=jax.ShapeDtypeStruct((_TROWS, _TD), jnp.float32),
    )(conv_w_pad, feat)


_NWORK = 32
_RPW = 1248
_XY_CH = 104
_STEPS = _RPW // _XY_CH
_ENT = _XY_CH * _NZ
_GCH = (128, 128, 128, 32)


@functools.cache
def _gather_combine_kernel():
    mesh = plsc.VectorSubcoreMesh(core_axis_name="c", subcore_axis_name="s")
    return functools.partial(
        pl.kernel,
        mesh=mesh,
        out_type=jax.ShapeDtypeStruct((_NXY, _COUT), jnp.float32),
        scratch_types=[
            pltpu.VMEM((_ENT,), jnp.int32),
            pltpu.VMEM((_ENT + 16,), jnp.float32),
            pltpu.VMEM((_ENT, _TD), jnp.float32),
            pltpu.VMEM((_XY_CH, _COUT), jnp.float32),
            pltpu.SemaphoreType.DMA,
        ],
    )(_gather_combine_body)


def _gather_combine_body(table_hbm, idx_hbm, w_hbm, out_hbm,
                         idx_v, w_v, rows_v, out_v, sem):
    wid = lax.axis_index("s") * 2 + lax.axis_index("c")
    row0 = wid * _RPW + jnp.maximum(wid - 24, 0) * 8

    def do_chunk(rowbase, nrows, gch):
        rowbase = pl.multiple_of(rowbase, 8)
        nent = nrows * _NZ
        ebase = rowbase * _NZ
        pltpu.sync_copy(idx_hbm.at[pl.ds(ebase, nent)],
                        idx_v.at[pl.ds(0, nent)])
        pltpu.sync_copy(w_hbm.at[pl.ds(ebase, nent)],
                        w_v.at[pl.ds(0, nent)])
        cops = []
        off = 0
        for g in gch:
            cops.append(pltpu.async_copy(
                table_hbm.at[idx_v.at[pl.ds(off, g)]],
                rows_v.at[pl.ds(off, g)], sem))
            off += g
        for c in cops:
            c.wait()

        def body(i, carry):
            wv = w_v[pl.ds(4 * i, 16)]
            for s in range(_COUT // 16):
                sl = pl.ds(s * 16, 16)
                acc = wv[0] * rows_v[4 * i, sl]
                acc = acc + wv[1] * rows_v[4 * i + 1, sl]
                acc = acc + wv[2] * rows_v[4 * i + 2, sl]
                acc = acc + wv[3] * rows_v[4 * i + 3, sl]
                out_v[i, sl] = acc
            return carry

        lax.fori_loop(0, nrows, body, 0)
        pltpu.sync_copy(out_v.at[pl.ds(0, nrows)],
                        out_hbm.at[pl.ds(rowbase, nrows)])

    for step in range(_STEPS):
        do_chunk(row0 + step * _XY_CH, _XY_CH, _GCH)

    @pl.when(wid >= 24)
    def _():
        do_chunk(row0 + _RPW, 8, (32,))


_CHD = 2000
_ND = _NXY // _CHD


def _stats_kernel(x_ref, out_ref):
    @pl.when(pl.program_id(0) == 0)
    def _():
        out_ref[...] = jnp.zeros_like(out_ref)

    xb = x_ref[...]
    out_ref[0:1, :] += jnp.sum(xb, axis=0, keepdims=True)
    out_ref[1:2, :] += jnp.sum(xb * xb, axis=0, keepdims=True)


def _stats(x):
    return pl.pallas_call(
        _stats_kernel,
        grid=(_ND,),
        in_specs=[pl.BlockSpec((_CHD, _COUT), lambda i: (i, 0))],
        out_specs=pl.BlockSpec((2, _COUT), lambda i: (0, 0)),
        out_shape=jax.ShapeDtypeStruct((2, _COUT), jnp.float32),
    )(x)


_CBN = 8


def _bn_kernel(x_ref, st_ref, g_ref, b_ref, eye_ref, out_ref):
    inv_n = jnp.float32(1.0 / _NXY)
    mean = st_ref[0:1, :] * inv_n
    var = st_ref[1:2, :] * inv_n - mean * mean
    scale = g_ref[...] / jnp.sqrt(var + 1e-5)
    eb = eye_ref[...]
    sb = eb * scale
    t = lax.dot_general(sb, x_ref[...], (((1,), (1,)), ((), ())),
                        preferred_element_type=jnp.float32)
    ms = lax.dot_general(eb, mean * scale, (((1,), (1,)), ((), ())),
                         preferred_element_type=jnp.float32)
    out_ref[...] = jnp.maximum(t - ms + b_ref[...], 0.0)


def _bn(x, st, gamma, beta, eye):
    return pl.pallas_call(
        _bn_kernel,
        grid=(_COUT // _CBN,),
        in_specs=[
            pl.BlockSpec((_NXY, _COUT), lambda i: (0, 0)),
            pl.BlockSpec((2, _COUT), lambda i: (0, 0)),
            pl.BlockSpec((1, _COUT), lambda i: (0, 0)),
            pl.BlockSpec((_CBN, 1), lambda i: (i, 0)),
            pl.BlockSpec((_CBN, _COUT), lambda i: (i, 0)),
        ],
        out_specs=pl.BlockSpec((_CBN, _NXY), lambda i: (i, 0)),
        out_shape=jax.ShapeDtypeStruct((_COUT, _NXY), jnp.float32),
    )(x, st, gamma, beta, eye)


def _get_vox_points():
    g = jnp.stack(jnp.meshgrid(jnp.arange(_NX), jnp.arange(_NY),
                               jnp.arange(_NZ), indexing='ij')).astype(jnp.float32)
    vs = jnp.array([0.5, 0.5, 1.5], jnp.float32)
    nv = jnp.array([_NX, _NY, _NZ], jnp.float32)
    origin = jnp.array([0.0, 0.0, -1.0], jnp.float32) - nv / 2.0 * vs
    return (g * vs.reshape(3, 1, 1, 1) + origin.reshape(3, 1, 1, 1)).reshape(1, 3, -1)


def kernel(mlvl_feats, points, ori_points, img, lidar2camera, lidar2image,
           cam_intrinsic, cam_2_lidar, img_aug_matrix, lidar_aug_matrix,
           img_metas, conv_w, conv_b, bn_gamma, bn_beta):
    feat = mlvl_feats[0].reshape(_NCAM, _CIN, _FH * _FW)
    la = lidar_aug_matrix[0]
    la_t = la[:3, -1]
    la_r = la[:3, :3]
    ia = img_aug_matrix[0]
    ia_t = ia[..., -1]
    ia_r = ia.at[..., -1].set(0.0)
    proj = jnp.matmul(ia_r, lidar2image[0])[:, :3, :]
    pt0 = _get_vox_points()
    pt = pt0 - la_t.reshape(1, 3, 1)
    pt = jnp.matmul(la_r.T, pt)
    pt = jnp.concatenate([pt, jnp.ones_like(pt[:, :1])], axis=1)
    pt = jnp.broadcast_to(pt, (_NCAM, 4, pt.shape[-1]))
    p2i = jnp.matmul(proj, pt).reshape(_NCAM, 3, _NX, _NY * _NZ)
    par = ia_t[:, :2].reshape(-1)
    p2 = jnp.transpose(points, (0, 2, 3, 1)).reshape(_NX, _NY * _NZ)

    idx2, w2 = _project(par, p2i, p2)
    idx = idx2.reshape(_NVOX)
    wvx = w2.reshape(_NVOX)
    cw_pad = jnp.zeros((_TD, _CIN), jnp.float32).at[:_COUT].set(conv_w)
    table = _make_table(cw_pad, feat)
    x = _gather_combine_kernel()(table, idx, wvx)
    st = _stats(x)
    out = _bn(x, st, bn_gamma.reshape(1, _COUT), bn_beta.reshape(_COUT, 1),
              jnp.eye(_COUT, dtype=jnp.float32))
    return out.reshape(1, _COUT, _NX, _NY)
